# baseline (speedup 1.0000x reference)
import jax
import jax.numpy as jnp
from jax import lax
from jax.experimental import pallas as pl
from jax.experimental.pallas import tpu as pltpu
from jax.experimental.pallas import tpu_sc as plsc

_D = 50
_DP = 56
_B = 4096
_S = 128
_NC = 2
_NS = 16
_NW = _NC * _NS
_N = _B * _S
_PER_W = _N // _NW
_GN = 512
_CPG = _GN // _S
_NG = _PER_W // _GN
_NB = _GN // 16


def _fire_gathers(table_hbm, idx_v, rows_v, gsem, g, buf):
    for i in range(_CPG):
        pltpu.async_copy(table_hbm.at[idx_v.at[g * _CPG + i]],
                         rows_v.at[buf, pl.ds(i * _S, _S)], gsem)


def _drain_gathers(table_hbm, idx_v, rows_v, gsem, buf):
    for i in range(_CPG):
        pltpu.make_async_copy(table_hbm.at[idx_v.at[0]],
                              rows_v.at[buf, pl.ds(i * _S, _S)], gsem).wait()


def _transpose(rows_v, outT_v, buf):
    iota = lax.iota(jnp.int32, 16)

    @plsc.parallel_loop(0, _NB, 1, unroll=8)
    def blk(n0):
        row_ids = n0 * 16 + iota
        for d0 in range(0, _D, 25):
            vals = [plsc.load_gather(rows_v.at[buf],
                                     [row_ids, jnp.full((16,), d, jnp.int32)])
                    for d in range(d0, d0 + 25)]
            for k, d in enumerate(range(d0, d0 + 25)):
                outT_v[buf, d, pl.ds(n0 * 16, 16)] = vals[k]


def _body(idx_hbm, table_hbm, out_hbm, idx_v, rows_v, outT_v, isem, gsem,
          wsem0, wsem1):
    wid = lax.axis_index("s") * _NC + lax.axis_index("c")
    wbase = wid * _PER_W
    wsems = (wsem0, wsem1)

    pltpu.async_copy(idx_hbm.at[pl.ds(wid * (_PER_W // _S), _PER_W // _S)],
                     idx_v, isem).wait()
    _fire_gathers(table_hbm, idx_v, rows_v, gsem, 0, 0)

    def pair(p, carry):
        for half in range(2):
            g = p * 2 + half
            _drain_gathers(table_hbm, idx_v, rows_v, gsem, half)
            if half == 0:
                _fire_gathers(table_hbm, idx_v, rows_v, gsem, g + 1, 1)
            else:
                @pl.when(p < _NG // 2 - 1)
                def _():
                    _fire_gathers(table_hbm, idx_v, rows_v, gsem, g + 1, 0)

            @pl.when(p > 0)
            def _():
                pltpu.make_async_copy(
                    outT_v.at[half], out_hbm.at[:, pl.ds(0, _GN)],
                    wsems[half]).wait()

            _transpose(rows_v, outT_v, half)
            pltpu.async_copy(outT_v.at[half],
                             out_hbm.at[:, pl.ds(wbase + g * _GN, _GN)],
                             wsems[half])
        return carry

    lax.fori_loop(0, _NG // 2, pair, 0)

    pltpu.make_async_copy(outT_v.at[0], out_hbm.at[:, pl.ds(0, _GN)],
                          wsems[0]).wait()
    pltpu.make_async_copy(outT_v.at[1], out_hbm.at[:, pl.ds(0, _GN)],
                          wsems[1]).wait()


@jax.jit
def _gather(idx, table):
    mesh = plsc.VectorSubcoreMesh(core_axis_name="c", subcore_axis_name="s")
    f = pl.kernel(
        _body,
        out_type=jax.ShapeDtypeStruct((_D, _N), jnp.float32),
        mesh=mesh,
        scratch_types=[
            pltpu.VMEM((_PER_W // _S, _S), jnp.int32),
            pltpu.VMEM((2, _GN, _DP), jnp.float32),
            pltpu.VMEM((2, _D, _GN), jnp.float32),
            pltpu.SemaphoreType.DMA,
            pltpu.SemaphoreType.DMA,
            pltpu.SemaphoreType.DMA,
            pltpu.SemaphoreType.DMA,
        ],
        compiler_params=pltpu.CompilerParams(use_tc_tiling_on_sc=False,
                                             needs_layout_passes=False),
    )
    return f(idx, table)


def kernel(input, table):
    idx = input.astype(jnp.int32)
    table_p = jnp.pad(table, ((0, 0), (0, _DP - _D)))
    out = _gather(idx, table_p)
    return jnp.transpose(out.reshape(_D, _B, _S), (1, 2, 0))


# speedup vs baseline: 1.0064x; 1.0064x over previous
import jax
import jax.numpy as jnp
from jax import lax
from jax.experimental import pallas as pl
from jax.experimental.pallas import tpu as pltpu
from jax.experimental.pallas import tpu_sc as plsc

_D = 50
_DP = 56
_B = 4096
_S = 128
_NC = 2
_NS = 16
_NW = _NC * _NS
_N = _B * _S
_PER_W = _N // _NW
_GN = 512
_CPG = _GN // _S
_NG = _PER_W // _GN
_NB = _GN // 16


def _fire_gathers(table_hbm, idx_v, rows_v, gsem, g, buf):
    for i in range(_CPG):
        pltpu.async_copy(table_hbm.at[idx_v.at[g * _CPG + i]],
                         rows_v.at[buf, pl.ds(i * _S, _S)], gsem)


def _drain_gathers(table_hbm, idx_v, rows_v, gsem, buf):
    for i in range(_CPG):
        pltpu.make_async_copy(table_hbm.at[idx_v.at[0]],
                              rows_v.at[buf, pl.ds(i * _S, _S)], gsem).wait()


def _transpose(rows_v, outT_v, buf):
    iota = lax.iota(jnp.int32, 16)

    @plsc.parallel_loop(0, _NB, 1, unroll=4)
    def blk(n0):
        row_ids = n0 * 16 + iota
        for d0 in range(0, _D, 25):
            vals = [plsc.load_gather(rows_v.at[buf],
                                     [row_ids, jnp.full((16,), d, jnp.int32)])
                    for d in range(d0, d0 + 25)]
            for k, d in enumerate(range(d0, d0 + 25)):
                outT_v[buf, d, pl.ds(n0 * 16, 16)] = vals[k]


def _body(idx_hbm, table_hbm, out_hbm, idx_v, rows_v, outT_v, isem, gsem,
          wsem0, wsem1):
    wid = lax.axis_index("s") * _NC + lax.axis_index("c")
    wbase = wid * _PER_W
    wsems = (wsem0, wsem1)

    pltpu.async_copy(idx_hbm.at[pl.ds(wid * (_PER_W // _S), _PER_W // _S)],
                     idx_v, isem).wait()
    _fire_gathers(table_hbm, idx_v, rows_v, gsem, 0, 0)

    def pair(p, carry):
        for half in range(2):
            g = p * 2 + half
            _drain_gathers(table_hbm, idx_v, rows_v, gsem, half)
            if half == 0:
                _fire_gathers(table_hbm, idx_v, rows_v, gsem, g + 1, 1)
            else:
                @pl.when(p < _NG // 2 - 1)
                def _():
                    _fire_gathers(table_hbm, idx_v, rows_v, gsem, g + 1, 0)

            @pl.when(p > 0)
            def _():
                pltpu.make_async_copy(
                    outT_v.at[half], out_hbm.at[:, pl.ds(0, _GN)],
                    wsems[half]).wait()

            _transpose(rows_v, outT_v, half)
            pltpu.async_copy(outT_v.at[half],
                             out_hbm.at[:, pl.ds(wbase + g * _GN, _GN)],
                             wsems[half])
        return carry

    lax.fori_loop(0, _NG // 2, pair, 0)

    pltpu.make_async_copy(outT_v.at[0], out_hbm.at[:, pl.ds(0, _GN)],
                          wsems[0]).wait()
    pltpu.make_async_copy(outT_v.at[1], out_hbm.at[:, pl.ds(0, _GN)],
                          wsems[1]).wait()


@jax.jit
def _gather(idx, table):
    mesh = plsc.VectorSubcoreMesh(core_axis_name="c", subcore_axis_name="s")
    f = pl.kernel(
        _body,
        out_type=jax.ShapeDtypeStruct((_D, _N), jnp.float32),
        mesh=mesh,
        scratch_types=[
            pltpu.VMEM((_PER_W // _S, _S), jnp.int32),
            pltpu.VMEM((2, _GN, _DP), jnp.float32),
            pltpu.VMEM((2, _D, _GN), jnp.float32),
            pltpu.SemaphoreType.DMA,
            pltpu.SemaphoreType.DMA,
            pltpu.SemaphoreType.DMA,
            pltpu.SemaphoreType.DMA,
        ],
        compiler_params=pltpu.CompilerParams(use_tc_tiling_on_sc=False,
                                             needs_layout_passes=False),
    )
    return f(idx, table)


def kernel(input, table):
    idx = input.astype(jnp.int32)
    table_p = jnp.pad(table, ((0, 0), (0, _DP - _D)))
    out = _gather(idx, table_p)
    return jnp.transpose(out.reshape(_D, _B, _S), (1, 2, 0))
